# baseline (speedup 1.0000x reference)
import jax
import jax.numpy as jnp
from jax.experimental import pallas as pl
from jax.experimental.pallas import tpu as pltpu


def _mlp(ws, h):
    for Wt, b in ws[:-1]:
        h = jnp.maximum(
            jnp.dot(h, Wt, preferred_element_type=jnp.float32) + b, 0.0)
    Wt, b = ws[-1]
    return jnp.dot(h, Wt, preferred_element_type=jnp.float32) + b


def _gnn_kernel(I, J, D, refs):
    k_ref = refs[0]
    x_ref, mmT_ref = refs[1], refs[2]
    wrefs, out_ref = refs[3:-1], refs[-1]
    ws = [[(wrefs[m * 8 + 2 * l][...], wrefs[m * 8 + 2 * l + 1][...])
           for l in range(4)] for m in range(4)]

    mmT = mmT_ref[...]
    init = x_ref[...]
    N = I * J

    max_T = jnp.max(mmT, axis=0, keepdims=True)
    pred_t = mmT - 1
    succ_t = mmT + 1
    has_p = mmT != 0
    has_s = mmT != max_T

    def step(_, x):
        xT = jnp.transpose(x, (1, 2, 0))
        pfT = jnp.zeros((J, D, I), jnp.float32)
        sfT = jnp.zeros((J, D, I), jnp.float32)
        pdone = jnp.zeros((J, 1, I), jnp.bool_)
        sdone = jnp.zeros((J, 1, I), jnp.bool_)
        for a in range(J):
            col = mmT[a:a + 1]
            xa = xT[a:a + 1]
            pm = (col == pred_t) & (~pdone)
            sm = (col == succ_t) & (~sdone)
            pfT = pfT + jnp.where(pm, xa, 0.0)
            sfT = sfT + jnp.where(sm, xa, 0.0)
            pdone = pdone | pm
            sdone = sdone | sm
        x0 = xT[0:1]
        pfT = jnp.where(pdone, pfT, x0)
        sfT = jnp.where(sdone, sfT, x0)
        a1_inT = jnp.where(has_p, pfT, 0.0)
        a2_inT = jnp.where(has_s, sfT, 0.0)

        a3_in = jnp.sum(x, axis=0, keepdims=True) - x
        xf = x.reshape(N, D)
        a4_vec = jnp.maximum(jnp.sum(xf, axis=0, keepdims=True), 0.0)

        a1 = jnp.maximum(
            _mlp(ws[0], jnp.transpose(a1_inT, (2, 0, 1)).reshape(N, D)), 0.0)
        a2 = jnp.maximum(
            _mlp(ws[1], jnp.transpose(a2_inT, (2, 0, 1)).reshape(N, D)), 0.0)
        a3 = jnp.maximum(_mlp(ws[2], a3_in.reshape(N, D)), 0.0)
        a4 = jnp.broadcast_to(a4_vec, (N, D))

        cat = jnp.concatenate(
            [a1, a2, a3, a4, xf, init.reshape(N, D)], axis=-1)
        return _mlp(ws[3], cat).reshape(I, J, D)

    out_ref[...] = jax.lax.fori_loop(0, k_ref[0], step, init)


def kernel(x, params, MM, PM, K):
    del PM
    I, J, D = x.shape
    wlist = []
    for name in ('f1', 'f2', 'f3', 'f4'):
        for (W, b) in params[name]:
            wlist.append(W.T)
            wlist.append(b.reshape(1, -1))
    MMT = MM.T[:, None, :]
    karr = jnp.asarray(K, jnp.int32).reshape(1)

    run = pl.pallas_call(
        lambda *refs: _gnn_kernel(I, J, D, refs),
        grid_spec=pltpu.PrefetchScalarGridSpec(
            num_scalar_prefetch=1,
            grid=(1,),
            in_specs=[pl.BlockSpec(a.shape, lambda *_, nd=a.ndim: (0,) * nd)
                      for a in (x, MMT, *wlist)],
            out_specs=pl.BlockSpec((I, J, D), lambda *_: (0, 0, 0)),
        ),
        out_shape=jax.ShapeDtypeStruct((I, J, D), jnp.float32),
    )
    return run(karr, x, MMT, *wlist)


# speedup vs baseline: 1.0647x; 1.0647x over previous
import jax
import jax.numpy as jnp
from jax.experimental import pallas as pl
from jax.experimental.pallas import tpu as pltpu

_SPLIT = 2


def _mlp(ws, h):
    for Wt, b in ws[:-1]:
        h = jnp.maximum(
            jnp.dot(h, Wt, preferred_element_type=jnp.float32) + b, 0.0)
    Wt, b = ws[-1]
    return jnp.dot(h, Wt, preferred_element_type=jnp.float32) + b


def _gnn_step(IB, J, D, refs):
    x_ref, xfull_ref, init_ref, mmT_ref = refs[0], refs[1], refs[2], refs[3]
    wrefs, out_ref = refs[4:-1], refs[-1]
    ws = [[(wrefs[m * 8 + 2 * l][...], wrefs[m * 8 + 2 * l + 1][...])
           for l in range(4)] for m in range(4)]

    x = x_ref[...]
    init = init_ref[...]
    mmT = mmT_ref[...][0]

    xT = jnp.transpose(x, (1, 2, 0))
    max_T = jnp.max(mmT, axis=0, keepdims=True)
    pred_t = mmT - 1
    succ_t = mmT + 1

    pfT = jnp.zeros((J, D, IB), jnp.float32)
    sfT = jnp.zeros((J, D, IB), jnp.float32)
    pdone = jnp.zeros((J, 1, IB), jnp.bool_)
    sdone = jnp.zeros((J, 1, IB), jnp.bool_)
    for a in range(J):
        col = mmT[a:a + 1]
        xa = xT[a:a + 1]
        pm = (col == pred_t) & (~pdone)
        sm = (col == succ_t) & (~sdone)
        pfT = pfT + jnp.where(pm, xa, 0.0)
        sfT = sfT + jnp.where(sm, xa, 0.0)
        pdone = pdone | pm
        sdone = sdone | sm
    x0 = xT[0:1]
    pfT = jnp.where(pdone, pfT, x0)
    sfT = jnp.where(sdone, sfT, x0)
    a1_inT = jnp.where(mmT != 0, pfT, 0.0)
    a2_inT = jnp.where(mmT != max_T, sfT, 0.0)

    xfull = xfull_ref[...]
    a3_in = jnp.sum(xfull, axis=0, keepdims=True) - x

    NB = IB * J
    xf = x.reshape(NB, D)
    xffull = xfull.reshape(xfull.shape[0] * J, D)
    a4_vec = jnp.maximum(jnp.sum(xffull, axis=0, keepdims=True), 0.0)

    a1 = jnp.maximum(
        _mlp(ws[0], jnp.transpose(a1_inT, (2, 0, 1)).reshape(NB, D)), 0.0)
    a2 = jnp.maximum(
        _mlp(ws[1], jnp.transpose(a2_inT, (2, 0, 1)).reshape(NB, D)), 0.0)
    a3 = jnp.maximum(_mlp(ws[2], a3_in.reshape(NB, D)), 0.0)
    a4 = jnp.broadcast_to(a4_vec, (NB, D))

    cat = jnp.concatenate([a1, a2, a3, a4, xf, init.reshape(NB, D)], axis=-1)
    out_ref[...] = _mlp(ws[3], cat).reshape(IB, J, D)


def kernel(x, params, MM, PM, K):
    del PM
    I, J, D = x.shape
    C = _SPLIT if I % _SPLIT == 0 else 1
    IB = I // C
    wlist = []
    for name in ('f1', 'f2', 'f3', 'f4'):
        for (W, b) in params[name]:
            wlist.append(W.T)
            wlist.append(b.reshape(1, -1))
    init = x
    MMT = jnp.transpose(MM.reshape(C, IB, J), (0, 2, 1))[:, :, None, :]

    wspecs = [pl.BlockSpec(w.shape, lambda c, nd=w.ndim: (0,) * nd)
              for w in wlist]
    step = pl.pallas_call(
        lambda *refs: _gnn_step(IB, J, D, refs),
        grid=(C,),
        in_specs=[
            pl.BlockSpec((IB, J, D), lambda c: (c, 0, 0)),
            pl.BlockSpec((I, J, D), lambda c: (0, 0, 0)),
            pl.BlockSpec((IB, J, D), lambda c: (c, 0, 0)),
            pl.BlockSpec((1, J, 1, IB), lambda c: (c, 0, 0, 0)),
            *wspecs,
        ],
        out_specs=pl.BlockSpec((IB, J, D), lambda c: (c, 0, 0)),
        out_shape=jax.ShapeDtypeStruct((I, J, D), jnp.float32),
        compiler_params=pltpu.CompilerParams(
            dimension_semantics=("parallel",)),
    )

    def body(_, xc):
        return step(xc, xc, init, MMT, *wlist)

    return jax.lax.fori_loop(0, K, body, x)


# speedup vs baseline: 1.3016x; 1.2225x over previous
import jax
import jax.numpy as jnp
from jax.experimental import pallas as pl
from jax.experimental.pallas import tpu as pltpu

_HID = 256


def _pack_weights(params, D):
    pieces, meta, row = [], {}, 0
    for name in ('f1', 'f2', 'f3', 'f4'):
        offs = []
        ps = params[name]
        for li, (W, b) in enumerate(ps):
            if li < len(ps) - 1:
                Wp = W.T
            else:
                Wp = W
            bp = b.reshape(1, -1)
            if bp.shape[1] < _HID:
                bp = jnp.pad(bp, ((0, 0), (0, _HID - bp.shape[1])))
            offs.append((row, Wp.shape[0]))
            pieces.append(Wp)
            row += Wp.shape[0]
            offs.append((row, 1))
            pieces.append(bp)
            row += 1
        meta[name] = offs
    return jnp.concatenate(pieces, axis=0), meta


def _mlp_packed(wp_ref, offs, h, out_dim):
    for li in range(3):
        (wr, wn), (br, _) = offs[2 * li], offs[2 * li + 1]
        W = wp_ref[wr:wr + wn, :]
        b = wp_ref[br:br + 1, :]
        h = jnp.maximum(
            jnp.dot(h, W, preferred_element_type=jnp.float32) + b, 0.0)
    (wr, wn), (br, _) = offs[6], offs[7]
    W = wp_ref[wr:wr + wn, :]
    b = wp_ref[br:br + 1, 0:out_dim]
    return jnp.dot(h, W.T, preferred_element_type=jnp.float32) + b


def _gnn_step(I, J, D, x, init, mmT, wp_ref, meta):
    max_T = jnp.max(mmT, axis=0, keepdims=True)
    pred_t = mmT - 1
    succ_t = mmT + 1

    xT = jnp.transpose(x, (1, 2, 0))
    pfT = jnp.zeros((J, D, I), jnp.float32)
    sfT = jnp.zeros((J, D, I), jnp.float32)
    pdone = jnp.zeros((J, 1, I), jnp.bool_)
    sdone = jnp.zeros((J, 1, I), jnp.bool_)
    for a in range(J):
        col = mmT[a:a + 1]
        xa = xT[a:a + 1]
        pm = (col == pred_t) & (~pdone)
        sm = (col == succ_t) & (~sdone)
        pfT = pfT + jnp.where(pm, xa, 0.0)
        sfT = sfT + jnp.where(sm, xa, 0.0)
        pdone = pdone | pm
        sdone = sdone | sm
    x0 = xT[0:1]
    pfT = jnp.where(pdone, pfT, x0)
    sfT = jnp.where(sdone, sfT, x0)
    a1_inT = jnp.where(mmT != 0, pfT, 0.0)
    a2_inT = jnp.where(mmT != max_T, sfT, 0.0)

    a3_in = jnp.sum(x, axis=0, keepdims=True) - x
    N = I * J
    xf = x.reshape(N, D)
    a4_vec = jnp.maximum(jnp.sum(xf, axis=0, keepdims=True), 0.0)

    a1 = jnp.maximum(_mlp_packed(
        wp_ref, meta['f1'],
        jnp.transpose(a1_inT, (2, 0, 1)).reshape(N, D), D), 0.0)
    a2 = jnp.maximum(_mlp_packed(
        wp_ref, meta['f2'],
        jnp.transpose(a2_inT, (2, 0, 1)).reshape(N, D), D), 0.0)
    a3 = jnp.maximum(_mlp_packed(wp_ref, meta['f3'], a3_in.reshape(N, D), D),
                     0.0)
    a4 = jnp.broadcast_to(a4_vec, (N, D))

    cat = jnp.concatenate([a1, a2, a3, a4, xf, init.reshape(N, D)], axis=-1)
    return _mlp_packed(wp_ref, meta['f4'], cat, D).reshape(I, J, D)


def kernel(x, params, MM, PM, K):
    del PM
    I, J, D = x.shape
    wp, meta = _pack_weights(params, D)
    MMT = MM.T[:, None, :]

    def _spec2_body(x_ref, mmT_ref, wp_ref, out_ref, xbuf_ref):
        k = pl.program_id(0)
        init = x_ref[...]

        @pl.when(k == 0)
        def _():
            xbuf_ref[...] = init

        new = _gnn_step(I, J, D, xbuf_ref[...], init, mmT_ref[...],
                        wp_ref, meta)
        xbuf_ref[...] = new
        out_ref[...] = new

    spec2 = pl.pallas_call(
        _spec2_body,
        grid=(2,),
        in_specs=[
            pl.BlockSpec((I, J, D), lambda k: (0, 0, 0)),
            pl.BlockSpec(MMT.shape, lambda k: (0, 0, 0)),
            pl.BlockSpec(wp.shape, lambda k: (0, 0)),
        ],
        out_specs=pl.BlockSpec((I, J, D), lambda k: (0, 0, 0)),
        out_shape=jax.ShapeDtypeStruct((I, J, D), jnp.float32),
        scratch_shapes=[pltpu.VMEM((I, J, D), jnp.float32)],
        compiler_params=pltpu.CompilerParams(
            dimension_semantics=("arbitrary",)),
    )

    def _iter_body(x_ref, init_ref, mmT_ref, wp_ref, out_ref):
        out_ref[...] = _gnn_step(I, J, D, x_ref[...], init_ref[...],
                                 mmT_ref[...], wp_ref, meta)

    one_iter = pl.pallas_call(
        _iter_body,
        out_shape=jax.ShapeDtypeStruct((I, J, D), jnp.float32),
    )

    def _generic(xx):
        return jax.lax.fori_loop(
            0, K, lambda _, xc: one_iter(xc, xx, MMT, wp), xx)

    return jax.lax.cond(jnp.asarray(K) == 2,
                        lambda xx: spec2(xx, MMT, wp),
                        _generic, x)


# speedup vs baseline: 1.3930x; 1.0702x over previous
import jax
import jax.numpy as jnp
from jax.experimental import pallas as pl
from jax.experimental.pallas import tpu as pltpu

_TRHS = (((1,), (1,)), ((), ()))


def _flatten_params(params):
    flat = []
    for name in ('f1', 'f2', 'f3', 'f4'):
        for W, b in params[name]:
            flat.append(W)
            flat.append(b.reshape(1, -1))
    return flat


def _mlp_refs(refs, h):
    for li in range(3):
        W = refs[2 * li][...]
        b = refs[2 * li + 1][...]
        h = jnp.maximum(
            jax.lax.dot_general(h, W, _TRHS,
                                preferred_element_type=jnp.float32) + b, 0.0)
    W = refs[6][...]
    b = refs[7][...]
    return jax.lax.dot_general(h, W, _TRHS,
                               preferred_element_type=jnp.float32) + b


def _gnn_step(I, J, D, x, init, mmT, prefs):
    max_T = jnp.max(mmT, axis=0, keepdims=True)
    pred_t = mmT - 1
    succ_t = mmT + 1

    xT = jnp.transpose(x, (1, 2, 0))
    pfT = jnp.zeros((J, D, I), jnp.float32)
    sfT = jnp.zeros((J, D, I), jnp.float32)
    pdone = jnp.zeros((J, 1, I), jnp.bool_)
    sdone = jnp.zeros((J, 1, I), jnp.bool_)
    for a in range(J):
        col = mmT[a:a + 1]
        xa = xT[a:a + 1]
        pm = (col == pred_t) & (~pdone)
        sm = (col == succ_t) & (~sdone)
        pfT = pfT + jnp.where(pm, xa, 0.0)
        sfT = sfT + jnp.where(sm, xa, 0.0)
        pdone = pdone | pm
        sdone = sdone | sm
    x0 = xT[0:1]
    pfT = jnp.where(pdone, pfT, x0)
    sfT = jnp.where(sdone, sfT, x0)
    a1_inT = jnp.where(mmT != 0, pfT, 0.0)
    a2_inT = jnp.where(mmT != max_T, sfT, 0.0)

    a3_in = jnp.sum(x, axis=0, keepdims=True) - x
    N = I * J
    xf = x.reshape(N, D)
    a4_vec = jnp.maximum(jnp.sum(xf, axis=0, keepdims=True), 0.0)

    a1 = jnp.maximum(_mlp_refs(
        prefs[0:8], jnp.transpose(a1_inT, (2, 0, 1)).reshape(N, D)), 0.0)
    a2 = jnp.maximum(_mlp_refs(
        prefs[8:16], jnp.transpose(a2_inT, (2, 0, 1)).reshape(N, D)), 0.0)
    a3 = jnp.maximum(_mlp_refs(prefs[16:24], a3_in.reshape(N, D)), 0.0)
    a4 = jnp.broadcast_to(a4_vec, (N, D))

    cat = jnp.concatenate([a1, a2, a3, a4, xf, init.reshape(N, D)], axis=-1)
    return _mlp_refs(prefs[24:32], cat).reshape(I, J, D)


def kernel(x, params, MM, PM, K):
    del PM
    I, J, D = x.shape
    flat = _flatten_params(params)
    MMT = MM.T[:, None, :]

    def _spec2_body(x_ref, mmT_ref, *rest):
        prefs, out_ref, xbuf_ref = rest[:32], rest[32], rest[33]
        k = pl.program_id(0)
        init = x_ref[...]

        @pl.when(k == 0)
        def _():
            xbuf_ref[...] = init

        new = _gnn_step(I, J, D, xbuf_ref[...], init, mmT_ref[...], prefs)
        xbuf_ref[...] = new
        out_ref[...] = new

    full = lambda s: pl.BlockSpec(s, lambda k: (0,) * len(s))
    spec2 = pl.pallas_call(
        _spec2_body,
        grid=(2,),
        in_specs=[full(x.shape), full(MMT.shape)] + [full(a.shape)
                                                     for a in flat],
        out_specs=full((I, J, D)),
        out_shape=jax.ShapeDtypeStruct((I, J, D), jnp.float32),
        scratch_shapes=[pltpu.VMEM((I, J, D), jnp.float32)],
        compiler_params=pltpu.CompilerParams(
            dimension_semantics=("arbitrary",)),
    )

    def _iter_body(x_ref, init_ref, mmT_ref, *rest):
        prefs, out_ref = rest[:32], rest[32]
        out_ref[...] = _gnn_step(I, J, D, x_ref[...], init_ref[...],
                                 mmT_ref[...], prefs)

    one_iter = pl.pallas_call(
        _iter_body,
        out_shape=jax.ShapeDtypeStruct((I, J, D), jnp.float32),
    )

    def _generic(xx):
        return jax.lax.fori_loop(
            0, K, lambda _, xc: one_iter(xc, xx, MMT, *flat), xx)

    return jax.lax.cond(jnp.asarray(K) == 2,
                        lambda xx: spec2(xx, MMT, *flat),
                        _generic, x)
